# Initial kernel scaffold; baseline (speedup 1.0000x reference)
import jax
import jax.numpy as jnp
from jax.experimental import pallas as pl


def kernel(feats, weights, table):
    raise NotImplementedError("write your pallas kernel here")


# speedup vs baseline: 2.8253x; 2.8253x over previous
import functools

import jax
import jax.numpy as jnp
from jax import lax
from jax.experimental import pallas as pl
from jax.experimental.pallas import tpu as pltpu
from jax.experimental.pallas import tpu_sc as plsc

_NC = 2
_NS = 16
_NW = _NC * _NS
_LANES = 16


def kernel(feats, weights, table):
    B, H = feats.shape
    V, D = table.shape
    CB = 2
    RB = B // _NW
    NCHUNK = RB // CB
    NBUF = 4
    RPC = CB * H

    HP = 64
    feats2 = feats.reshape(B // CB, RPC).astype(jnp.int32)
    weights2 = jnp.pad(weights, ((0, 0), (0, HP - H))).reshape(B // CB, CB * HP)

    mesh = plsc.VectorSubcoreMesh(core_axis_name="c", subcore_axis_name="s")

    @functools.partial(
        pl.kernel,
        out_type=jax.ShapeDtypeStruct((B, D), jnp.float32),
        mesh=mesh,
        scratch_types=[
            pltpu.VMEM((NCHUNK, RPC), jnp.int32),
            pltpu.VMEM((NCHUNK, CB * HP), jnp.float32),
            pltpu.VMEM((NBUF, RPC, D), jnp.float32),
            pltpu.VMEM((RB, D), jnp.float32),
            pltpu.SemaphoreType.DMA,
        ],
        compiler_params=pltpu.CompilerParams(use_tc_tiling_on_sc=False),
    )
    def run(feats_hbm, w_hbm, table_hbm, out_hbm, idx_v, w_v, rows_v, out_v, sem):
        wid = lax.axis_index("s") * _NC + lax.axis_index("c")
        chunk0 = wid * NCHUNK

        pltpu.sync_copy(feats_hbm.at[pl.ds(chunk0, NCHUNK)], idx_v)
        pltpu.sync_copy(w_hbm.at[pl.ds(chunk0, NCHUNK)], w_v)

        def fire(g, b):
            pltpu.async_copy(table_hbm.at[idx_v.at[g]], rows_v.at[b], sem)

        def wait(g, b):
            pltpu.make_async_copy(
                table_hbm.at[idx_v.at[g]], rows_v.at[b], sem).wait()

        def compute(g, b):
            for cb in range(CB):
                wvecs = [w_v[g, pl.ds(cb * HP + o, _LANES)] for o in (0, 16, 32, 40)]
                acc0 = jnp.zeros((_LANES,), jnp.float32)
                acc1 = jnp.zeros((_LANES,), jnp.float32)
                for l in range(H):
                    r = cb * H + l
                    w = wvecs[l // 16][l % 16] if l < 48 else wvecs[3][l - 40]
                    acc0 = acc0 + w * rows_v[b, r, pl.ds(0, _LANES)]
                    acc1 = acc1 + w * rows_v[b, r, pl.ds(_LANES, _LANES)]
                out_v[g * CB + cb, pl.ds(0, _LANES)] = acc0
                out_v[g * CB + cb, pl.ds(_LANES, _LANES)] = acc1

        for b in range(NBUF):
            fire(b, b)

        @pl.loop(0, NCHUNK - NBUF, step=NBUF)
        def _(g0):
            for b in range(NBUF):
                g = g0 + b
                wait(g, b)
                compute(g, b)
                fire(g + NBUF, b)

        for b in range(NBUF):
            g = NCHUNK - NBUF + b
            wait(g, b)
            compute(g, b)

        pltpu.sync_copy(out_v, out_hbm.at[pl.ds(wid * RB, RB)])

    return run(feats2, weights2, table)
